# Initial kernel scaffold; baseline (speedup 1.0000x reference)
import jax
import jax.numpy as jnp
from jax.experimental import pallas as pl


def kernel(x, edge_index, batch, Wl1, Wr1, b1, g1, bt1, Wl2, Wr2, b2, g2, bt2, Wl3, Wr3, b3, g3, bt3):
    raise NotImplementedError("write your pallas kernel here")


# speedup vs baseline: 7.5085x; 7.5085x over previous
import functools

import jax
import jax.numpy as jnp
from jax import lax
from jax.experimental import pallas as pl
from jax.experimental.pallas import tpu as pltpu
from jax.experimental.pallas import tpu_sc as plsc

N = 10000
E = 320000
D = 128
G = 16
EPS = 1e-5

NC = 2
NS = 16
K = 80
NP = 10240
ROWS_PER_TILE = NP // NS
CHUNKS = E // (NC * NS * K)
IB = 25
STAGES = CHUNKS // IB
F32 = jnp.float32


def _make_sc_scatter(with_count: bool):
    out_type = [jax.ShapeDtypeStruct((NC, NP, D), F32)]
    scratch = [
        pltpu.VMEM((IB, K), jnp.int32),
        pltpu.VMEM((IB, K), jnp.int32),
        pltpu.VMEM((K, D), F32),
        pltpu.VMEM_SHARED((NP, D), F32),
        pltpu.SemaphoreType.DMA,
    ]
    if with_count:
        out_type.append(jax.ShapeDtypeStruct((NC * NS, NP // 128, 128), F32))
        scratch.append(pltpu.VMEM((NP // 128, 128), F32))

    mesh = plsc.VectorSubcoreMesh(core_axis_name="c", subcore_axis_name="s")

    NZ = ROWS_PER_TILE // K

    def body(h, src_i, dst_i, zrow, *rest):
        if with_count:
            (out, cnt_out, src_v, dst_v, rows_v, acc_sp, sem, cnt_v) = rest
        else:
            (out, src_v, dst_v, rows_v, acc_sp, sem) = rest
        c = lax.axis_index("c")
        s = lax.axis_index("s")
        r0 = s * ROWS_PER_TILE
        pltpu.sync_copy(zrow, rows_v)
        if with_count:
            pltpu.sync_copy(zrow, cnt_v)

        def zstep(i, carry):
            pltpu.sync_copy(rows_v, acc_sp.at[pl.ds(r0 + i * K, K)])
            return carry

        lax.fori_loop(0, NZ, zstep, 0)
        w = c * NS + s
        plsc.subcore_barrier()
        ones16 = jnp.full((16,), 1.0, F32)

        def stage(t, carry):
            pltpu.sync_copy(src_i.at[w * STAGES + t], src_v)
            pltpu.sync_copy(dst_i.at[w * STAGES + t], dst_v)

            def step(j, c2):
                pltpu.async_copy(h.at[src_v.at[j]], rows_v, sem).wait()
                pltpu.sync_copy(rows_v, acc_sp.at[dst_v.at[j]], add=True)
                if with_count:
                    for g in range(K // 16):
                        dv = dst_v[j, pl.ds(g * 16, 16)]
                        plsc.addupdate_scatter(
                            cnt_v, [dv >> 7, dv & 127], ones16)
                return c2

            return lax.fori_loop(0, IB, step, carry)

        lax.fori_loop(0, STAGES, stage, 0)
        if with_count:
            pltpu.sync_copy(cnt_v, cnt_out.at[w])
        plsc.subcore_barrier()

        def wstep(i, carry):
            pltpu.sync_copy(acc_sp.at[pl.ds(r0 + i * K, K)], rows_v)
            pltpu.sync_copy(rows_v, out.at[c, pl.ds(r0 + i * K, K)])
            return carry

        lax.fori_loop(0, NZ, wstep, 0)

    return pl.kernel(
        body, out_type=out_type, mesh=mesh, scratch_types=scratch,
        compiler_params=pltpu.CompilerParams(needs_layout_passes=False))


_SC_SCATTER_CNT = _make_sc_scatter(True)
_SC_SCATTER = _make_sc_scatter(False)


def _tc_pre(x, wl, wr, b):
    def body(x_ref, wl_ref, wr_ref, b_ref, h_ref, r_ref):
        xv = x_ref[...]
        h_ref[...] = jnp.dot(xv, wl_ref[...], preferred_element_type=F32)
        r_ref[...] = (jnp.dot(xv, wr_ref[...], preferred_element_type=F32)
                      + b_ref[...])
    return pl.pallas_call(
        body,
        out_shape=[jax.ShapeDtypeStruct((N, D), F32),
                   jax.ShapeDtypeStruct((N, D), F32)],
    )(x, wl, wr, b.reshape(1, D))


def _bn_relu(y, gamma, beta):
    mu = jnp.mean(y, axis=0, keepdims=True)
    var = jnp.mean(y * y, axis=0, keepdims=True) - mu * mu
    xn = (y - mu) * lax.rsqrt(var + EPS)
    return jnp.maximum(gamma * xn + beta, 0.0)


def _tc_cnt(cnt_parts):
    def body(c_ref, o_ref):
        o_ref[...] = jnp.maximum(jnp.sum(c_ref[...], axis=0), 1.0)
    return pl.pallas_call(
        body,
        out_shape=jax.ShapeDtypeStruct((NP // 128, 128), F32),
    )(cnt_parts)


def _tc_mid(acc, denom, r, gamma, beta, wl, wr, b):
    def body(acc_ref, c_ref, r_ref, g_ref, bt_ref, wl_ref, wr_ref, b_ref,
             x_ref, h_ref, rn_ref):
        a = acc_ref[...]
        y = (a[0, :N] + a[1, :N]) / c_ref[...] + r_ref[...]
        xo = _bn_relu(y, g_ref[...], bt_ref[...])
        x_ref[...] = xo
        h_ref[...] = jnp.dot(xo, wl_ref[...], preferred_element_type=F32)
        rn_ref[...] = (jnp.dot(xo, wr_ref[...], preferred_element_type=F32)
                       + b_ref[...])
    return pl.pallas_call(
        body,
        out_shape=[jax.ShapeDtypeStruct((N, D), F32),
                   jax.ShapeDtypeStruct((N, D), F32),
                   jax.ShapeDtypeStruct((N, D), F32)],
    )(acc, denom, r, gamma.reshape(1, D), beta.reshape(1, D), wl, wr,
      b.reshape(1, D))


def _tc_final(acc, denom, r, gamma, beta, x1, x2, batch2d):
    def body(acc_ref, c_ref, r_ref, g_ref, bt_ref, x1_ref, x2_ref, b_ref,
             o_ref):
        a = acc_ref[...]
        y = (a[0, :N] + a[1, :N]) / c_ref[...] + r_ref[...]
        x3 = _bn_relu(y, g_ref[...], bt_ref[...])
        iota = lax.broadcasted_iota(jnp.int32, (G, N), 0)
        oh = (iota == b_ref[...]).astype(F32)
        p1 = jnp.dot(oh, x1_ref[...], preferred_element_type=F32)
        p2 = jnp.dot(oh, x2_ref[...], preferred_element_type=F32)
        p3 = jnp.dot(oh, x3, preferred_element_type=F32)
        counts = jnp.sum(oh, axis=1, keepdims=True)
        o_ref[...] = (jnp.concatenate([p1, p2, p3], axis=1)
                      / jnp.maximum(counts, 1.0))
    return pl.pallas_call(
        body,
        out_shape=jax.ShapeDtypeStruct((G, 3 * D), F32),
    )(acc, denom, r, gamma.reshape(1, D), beta.reshape(1, D), x1, x2,
      batch2d)


def kernel(x, edge_index, batch, Wl1, Wr1, b1, g1, bt1, Wl2, Wr2, b2, g2,
           bt2, Wl3, Wr3, b3, g3, bt3):
    src2 = edge_index[0].reshape(NC * NS * STAGES, IB, K)
    dst2 = edge_index[1].reshape(NC * NS * STAGES, IB, K)
    zrow = jnp.zeros((K, D), F32)
    batch2d = batch.reshape(1, N)

    h1, r1 = _tc_pre(x, Wl1, Wr1, b1)
    acc1, cnt_parts = _SC_SCATTER_CNT(h1, src2, dst2, zrow)
    denom = _tc_cnt(cnt_parts).reshape(NP, 1)[:N]
    x1, h2, r2 = _tc_mid(acc1, denom, r1, g1, bt1, Wl2, Wr2, b2)
    (acc2,) = _SC_SCATTER(h2, src2, dst2, zrow)
    x2, h3, r3 = _tc_mid(acc2, denom, r2, g2, bt2, Wl3, Wr3, b3)
    (acc3,) = _SC_SCATTER(h3, src2, dst2, zrow)
    return _tc_final(acc3, denom, r3, g3, bt3, x1, x2, batch2d)


# speedup vs baseline: 11.4332x; 1.5227x over previous
import functools

import jax
import jax.numpy as jnp
from jax import lax
from jax.experimental import pallas as pl
from jax.experimental.pallas import tpu as pltpu
from jax.experimental.pallas import tpu_sc as plsc

N = 10000
E = 320000
D = 128
G = 16
EPS = 1e-5

NC = 2
NS = 16
K = 80
NP = 10240
ROWS_PER_TILE = NP // NS
CHUNKS = E // (NC * NS * K)
IB = 25
STAGES = CHUNKS // IB
F32 = jnp.float32


def _make_sc_scatter(with_count: bool):
    out_type = [jax.ShapeDtypeStruct((NC, NP, D), F32)]
    scratch = [
        pltpu.VMEM((IB, K), jnp.int32),
        pltpu.VMEM((IB, K), jnp.int32),
        pltpu.VMEM((2, K, D), F32),
        pltpu.VMEM_SHARED((NP, D), F32),
        pltpu.SemaphoreType.DMA((2,)),
    ]
    if with_count:
        out_type.append(jax.ShapeDtypeStruct((NC * NS, NP // 128, 128), F32))
        scratch.append(pltpu.VMEM((NP // 128, 128), F32))

    mesh = plsc.VectorSubcoreMesh(core_axis_name="c", subcore_axis_name="s")

    NZ = ROWS_PER_TILE // K

    def body(h, src_i, dst_i, zrow, *rest):
        if with_count:
            (out, cnt_out, src_v, dst_v, rows_v, acc_sp, sem, cnt_v) = rest
        else:
            (out, src_v, dst_v, rows_v, acc_sp, sem) = rest
        c = lax.axis_index("c")
        s = lax.axis_index("s")
        r0 = s * ROWS_PER_TILE
        pltpu.sync_copy(zrow, rows_v.at[0])
        if with_count:
            pltpu.sync_copy(zrow, cnt_v)

        def zstep(i, carry):
            pltpu.sync_copy(rows_v.at[0], acc_sp.at[pl.ds(r0 + i * K, K)])
            return carry

        lax.fori_loop(0, NZ, zstep, 0)
        w = c * NS + s
        plsc.subcore_barrier()
        ones16 = jnp.full((16,), 1.0, F32)

        def stage(t, carry):
            pltpu.sync_copy(src_i.at[w * STAGES + t], src_v)
            pltpu.sync_copy(dst_i.at[w * STAGES + t], dst_v)
            pltpu.async_copy(h.at[src_v.at[0]], rows_v.at[0], sem.at[0])

            def step(j, c2):
                slot = lax.rem(j, 2)
                nxt = lax.rem(j + 1, 2)

                @pl.when(j + 1 < IB)
                def _fire():
                    pltpu.async_copy(h.at[src_v.at[j + 1]], rows_v.at[nxt],
                                     sem.at[nxt])

                pltpu.make_async_copy(h.at[src_v.at[j]], rows_v.at[slot],
                                      sem.at[slot]).wait()
                pltpu.sync_copy(rows_v.at[slot], acc_sp.at[dst_v.at[j]],
                                add=True)
                if with_count:
                    for g in range(K // 16):
                        dv = dst_v[j, pl.ds(g * 16, 16)]
                        plsc.addupdate_scatter(
                            cnt_v, [dv >> 7, dv & 127], ones16)
                return c2

            return lax.fori_loop(0, IB, step, carry)

        lax.fori_loop(0, STAGES, stage, 0)
        if with_count:
            pltpu.sync_copy(cnt_v, cnt_out.at[w])
        plsc.subcore_barrier()

        def wstep(i, carry):
            slot = lax.rem(i, 2)
            pltpu.sync_copy(acc_sp.at[pl.ds(r0 + i * K, K)], rows_v.at[slot])
            pltpu.sync_copy(rows_v.at[slot], out.at[c, pl.ds(r0 + i * K, K)])
            return carry

        lax.fori_loop(0, NZ, wstep, 0)

    return pl.kernel(
        body, out_type=out_type, mesh=mesh, scratch_types=scratch,
        compiler_params=pltpu.CompilerParams(needs_layout_passes=False))


_SC_SCATTER_CNT = _make_sc_scatter(True)
_SC_SCATTER = _make_sc_scatter(False)


def _tc_pre(x, wl, wr, b):
    def body(x_ref, wl_ref, wr_ref, b_ref, h_ref, r_ref):
        xv = x_ref[...]
        h_ref[...] = jnp.dot(xv, wl_ref[...], preferred_element_type=F32)
        r_ref[...] = (jnp.dot(xv, wr_ref[...], preferred_element_type=F32)
                      + b_ref[...])
    return pl.pallas_call(
        body,
        out_shape=[jax.ShapeDtypeStruct((N, D), F32),
                   jax.ShapeDtypeStruct((N, D), F32)],
    )(x, wl, wr, b.reshape(1, D))


def _bn_relu(y, gamma, beta):
    mu = jnp.mean(y, axis=0, keepdims=True)
    var = jnp.mean(y * y, axis=0, keepdims=True) - mu * mu
    xn = (y - mu) * lax.rsqrt(var + EPS)
    return jnp.maximum(gamma * xn + beta, 0.0)


def _tc_cnt(cnt_parts):
    def body(c_ref, o_ref):
        o_ref[...] = jnp.maximum(jnp.sum(c_ref[...], axis=0), 1.0)
    return pl.pallas_call(
        body,
        out_shape=jax.ShapeDtypeStruct((NP // 128, 128), F32),
    )(cnt_parts)


def _tc_mid(acc, denom, r, gamma, beta, wl, wr, b):
    def body(acc_ref, c_ref, r_ref, g_ref, bt_ref, wl_ref, wr_ref, b_ref,
             x_ref, h_ref, rn_ref):
        a = acc_ref[...]
        y = (a[0, :N] + a[1, :N]) / c_ref[...] + r_ref[...]
        xo = _bn_relu(y, g_ref[...], bt_ref[...])
        x_ref[...] = xo
        h_ref[...] = jnp.dot(xo, wl_ref[...], preferred_element_type=F32)
        rn_ref[...] = (jnp.dot(xo, wr_ref[...], preferred_element_type=F32)
                       + b_ref[...])
    return pl.pallas_call(
        body,
        out_shape=[jax.ShapeDtypeStruct((N, D), F32),
                   jax.ShapeDtypeStruct((N, D), F32),
                   jax.ShapeDtypeStruct((N, D), F32)],
    )(acc, denom, r, gamma.reshape(1, D), beta.reshape(1, D), wl, wr,
      b.reshape(1, D))


def _tc_final(acc, denom, r, gamma, beta, x1, x2, batch2d):
    def body(acc_ref, c_ref, r_ref, g_ref, bt_ref, x1_ref, x2_ref, b_ref,
             o_ref):
        a = acc_ref[...]
        y = (a[0, :N] + a[1, :N]) / c_ref[...] + r_ref[...]
        x3 = _bn_relu(y, g_ref[...], bt_ref[...])
        iota = lax.broadcasted_iota(jnp.int32, (G, N), 0)
        oh = (iota == b_ref[...]).astype(F32)
        p1 = jnp.dot(oh, x1_ref[...], preferred_element_type=F32)
        p2 = jnp.dot(oh, x2_ref[...], preferred_element_type=F32)
        p3 = jnp.dot(oh, x3, preferred_element_type=F32)
        counts = jnp.sum(oh, axis=1, keepdims=True)
        o_ref[...] = (jnp.concatenate([p1, p2, p3], axis=1)
                      / jnp.maximum(counts, 1.0))
    return pl.pallas_call(
        body,
        out_shape=jax.ShapeDtypeStruct((G, 3 * D), F32),
    )(acc, denom, r, gamma.reshape(1, D), beta.reshape(1, D), x1, x2,
      batch2d)


def kernel(x, edge_index, batch, Wl1, Wr1, b1, g1, bt1, Wl2, Wr2, b2, g2,
           bt2, Wl3, Wr3, b3, g3, bt3):
    src2 = edge_index[0].reshape(NC * NS * STAGES, IB, K)
    dst2 = edge_index[1].reshape(NC * NS * STAGES, IB, K)
    zrow = jnp.zeros((K, D), F32)
    batch2d = batch.reshape(1, N)

    h1, r1 = _tc_pre(x, Wl1, Wr1, b1)
    acc1, cnt_parts = _SC_SCATTER_CNT(h1, src2, dst2, zrow)
    denom = _tc_cnt(cnt_parts).reshape(NP, 1)[:N]
    x1, h2, r2 = _tc_mid(acc1, denom, r1, g1, bt1, Wl2, Wr2, b2)
    (acc2,) = _SC_SCATTER(h2, src2, dst2, zrow)
    x2, h3, r3 = _tc_mid(acc2, denom, r2, g2, bt2, Wl3, Wr3, b3)
    (acc3,) = _SC_SCATTER(h3, src2, dst2, zrow)
    return _tc_final(acc3, denom, r3, g3, bt3, x1, x2, batch2d)


# speedup vs baseline: 12.2148x; 1.0684x over previous
import functools

import jax
import jax.numpy as jnp
from jax import lax
from jax.experimental import pallas as pl
from jax.experimental.pallas import tpu as pltpu
from jax.experimental.pallas import tpu_sc as plsc

N = 10000
E = 320000
D = 128
G = 16
EPS = 1e-5

NC = 2
NS = 16
K = 80
NP = 10240
ROWS_PER_TILE = NP // NS
CHUNKS = E // (NC * NS * K)
IB = 25
STAGES = CHUNKS // IB
F32 = jnp.float32


def _make_sc_scatter(with_count: bool):
    NB = 2 if with_count else 4
    out_type = [jax.ShapeDtypeStruct((NC, NP, D), F32)]
    scratch = [
        pltpu.VMEM((IB, K), jnp.int32),
        pltpu.VMEM((IB, K), jnp.int32),
        pltpu.VMEM((NB, K, D), F32),
        pltpu.VMEM_SHARED((NP, D), F32),
        pltpu.SemaphoreType.DMA((NB,)),
        pltpu.SemaphoreType.DMA((NB,)),
    ]
    if with_count:
        out_type.append(jax.ShapeDtypeStruct((NC * NS, NP // 128, 128), F32))
        scratch.append(pltpu.VMEM((NP // 128, 128), F32))

    mesh = plsc.VectorSubcoreMesh(core_axis_name="c", subcore_axis_name="s")

    NZ = ROWS_PER_TILE // K

    def body(h, src_i, dst_i, zrow, *rest):
        if with_count:
            (out, cnt_out, src_v, dst_v, rows_v, acc_sp, sem, ssem,
             cnt_v) = rest
        else:
            (out, src_v, dst_v, rows_v, acc_sp, sem, ssem) = rest
        c = lax.axis_index("c")
        s = lax.axis_index("s")
        r0 = s * ROWS_PER_TILE
        pltpu.sync_copy(zrow, rows_v.at[0])
        if with_count:
            pltpu.sync_copy(zrow, cnt_v)

        def zstep(i, carry):
            pltpu.sync_copy(rows_v.at[0], acc_sp.at[pl.ds(r0 + i * K, K)])
            return carry

        lax.fori_loop(0, NZ, zstep, 0)
        w = c * NS + s
        plsc.subcore_barrier()
        ones16 = jnp.full((16,), 1.0, F32)

        def stage(t, carry):
            pltpu.sync_copy(src_i.at[w * STAGES + t], src_v)
            pltpu.sync_copy(dst_i.at[w * STAGES + t], dst_v)
            pltpu.async_copy(h.at[src_v.at[0]], rows_v.at[0], sem.at[0])
            if NB > 2:
                pltpu.async_copy(h.at[src_v.at[1]], rows_v.at[1], sem.at[1])

            def step(j, c2):
                slot = lax.rem(j, NB)
                if NB == 2:
                    nxt = lax.rem(j + 1, NB)

                    @pl.when(j + 1 < IB)
                    def _fire():
                        pltpu.async_copy(h.at[src_v.at[j + 1]],
                                         rows_v.at[nxt], sem.at[nxt])

                    pltpu.make_async_copy(h.at[src_v.at[j]],
                                          rows_v.at[slot],
                                          sem.at[slot]).wait()
                    pltpu.sync_copy(rows_v.at[slot], acc_sp.at[dst_v.at[j]],
                                    add=True)
                else:
                    tgt = lax.rem(j + 2, NB)

                    @pl.when(j >= 2)
                    def _drain():
                        pltpu.make_async_copy(zrow, rows_v.at[tgt],
                                              ssem.at[tgt]).wait()

                    @pl.when(j + 2 < IB)
                    def _fire():
                        pltpu.async_copy(h.at[src_v.at[j + 2]],
                                         rows_v.at[tgt], sem.at[tgt])

                    pltpu.make_async_copy(h.at[src_v.at[j]],
                                          rows_v.at[slot],
                                          sem.at[slot]).wait()
                    pltpu.async_copy(rows_v.at[slot],
                                     acc_sp.at[dst_v.at[j]],
                                     ssem.at[slot], add=True)
                if with_count:
                    for g in range(K // 16):
                        dv = dst_v[j, pl.ds(g * 16, 16)]
                        plsc.addupdate_scatter(
                            cnt_v, [dv >> 7, dv & 127], ones16)
                return c2

            carry = lax.fori_loop(0, IB, step, carry)
            if NB > 2:
                pltpu.make_async_copy(
                    zrow, rows_v.at[(IB - 2) % NB],
                    ssem.at[(IB - 2) % NB]).wait()
                pltpu.make_async_copy(
                    zrow, rows_v.at[(IB - 1) % NB],
                    ssem.at[(IB - 1) % NB]).wait()
            return carry

        lax.fori_loop(0, STAGES, stage, 0)
        if with_count:
            pltpu.sync_copy(cnt_v, cnt_out.at[w])
        plsc.subcore_barrier()

        def wstep(i, carry):
            slot = lax.rem(i, 2)
            pltpu.sync_copy(acc_sp.at[pl.ds(r0 + i * K, K)], rows_v.at[slot])
            pltpu.sync_copy(rows_v.at[slot], out.at[c, pl.ds(r0 + i * K, K)])
            return carry

        lax.fori_loop(0, NZ, wstep, 0)

    return pl.kernel(
        body, out_type=out_type, mesh=mesh, scratch_types=scratch,
        compiler_params=pltpu.CompilerParams(needs_layout_passes=False))


_SC_SCATTER_CNT = _make_sc_scatter(True)
_SC_SCATTER = _make_sc_scatter(False)


def _tc_pre(x, wl, wr, b):
    def body(x_ref, wl_ref, wr_ref, b_ref, h_ref, r_ref):
        xv = x_ref[...]
        h_ref[...] = jnp.dot(xv, wl_ref[...], preferred_element_type=F32)
        r_ref[...] = (jnp.dot(xv, wr_ref[...], preferred_element_type=F32)
                      + b_ref[...])
    return pl.pallas_call(
        body,
        out_shape=[jax.ShapeDtypeStruct((N, D), F32),
                   jax.ShapeDtypeStruct((N, D), F32)],
    )(x, wl, wr, b.reshape(1, D))


def _bn_relu(y, gamma, beta):
    mu = jnp.mean(y, axis=0, keepdims=True)
    var = jnp.mean(y * y, axis=0, keepdims=True) - mu * mu
    xn = (y - mu) * lax.rsqrt(var + EPS)
    return jnp.maximum(gamma * xn + beta, 0.0)


def _tc_cnt(cnt_parts):
    def body(c_ref, o_ref):
        o_ref[...] = jnp.maximum(jnp.sum(c_ref[...], axis=0), 1.0)
    return pl.pallas_call(
        body,
        out_shape=jax.ShapeDtypeStruct((NP // 128, 128), F32),
    )(cnt_parts)


def _tc_mid(acc, denom, r, gamma, beta, wl, wr, b):
    def body(acc_ref, c_ref, r_ref, g_ref, bt_ref, wl_ref, wr_ref, b_ref,
             x_ref, h_ref, rn_ref):
        a = acc_ref[...]
        y = (a[0, :N] + a[1, :N]) / c_ref[...] + r_ref[...]
        xo = _bn_relu(y, g_ref[...], bt_ref[...])
        x_ref[...] = xo
        h_ref[...] = jnp.dot(xo, wl_ref[...], preferred_element_type=F32)
        rn_ref[...] = (jnp.dot(xo, wr_ref[...], preferred_element_type=F32)
                       + b_ref[...])
    return pl.pallas_call(
        body,
        out_shape=[jax.ShapeDtypeStruct((N, D), F32),
                   jax.ShapeDtypeStruct((N, D), F32),
                   jax.ShapeDtypeStruct((N, D), F32)],
    )(acc, denom, r, gamma.reshape(1, D), beta.reshape(1, D), wl, wr,
      b.reshape(1, D))


def _tc_final(acc, denom, r, gamma, beta, x1, x2, batch2d):
    def body(acc_ref, c_ref, r_ref, g_ref, bt_ref, x1_ref, x2_ref, b_ref,
             o_ref):
        a = acc_ref[...]
        y = (a[0, :N] + a[1, :N]) / c_ref[...] + r_ref[...]
        x3 = _bn_relu(y, g_ref[...], bt_ref[...])
        iota = lax.broadcasted_iota(jnp.int32, (G, N), 0)
        oh = (iota == b_ref[...]).astype(F32)
        p1 = jnp.dot(oh, x1_ref[...], preferred_element_type=F32)
        p2 = jnp.dot(oh, x2_ref[...], preferred_element_type=F32)
        p3 = jnp.dot(oh, x3, preferred_element_type=F32)
        counts = jnp.sum(oh, axis=1, keepdims=True)
        o_ref[...] = (jnp.concatenate([p1, p2, p3], axis=1)
                      / jnp.maximum(counts, 1.0))
    return pl.pallas_call(
        body,
        out_shape=jax.ShapeDtypeStruct((G, 3 * D), F32),
    )(acc, denom, r, gamma.reshape(1, D), beta.reshape(1, D), x1, x2,
      batch2d)


def kernel(x, edge_index, batch, Wl1, Wr1, b1, g1, bt1, Wl2, Wr2, b2, g2,
           bt2, Wl3, Wr3, b3, g3, bt3):
    src2 = edge_index[0].reshape(NC * NS * STAGES, IB, K)
    dst2 = edge_index[1].reshape(NC * NS * STAGES, IB, K)
    zrow = jnp.zeros((K, D), F32)
    batch2d = batch.reshape(1, N)

    h1, r1 = _tc_pre(x, Wl1, Wr1, b1)
    acc1, cnt_parts = _SC_SCATTER_CNT(h1, src2, dst2, zrow)
    denom = _tc_cnt(cnt_parts).reshape(NP, 1)[:N]
    x1, h2, r2 = _tc_mid(acc1, denom, r1, g1, bt1, Wl2, Wr2, b2)
    (acc2,) = _SC_SCATTER(h2, src2, dst2, zrow)
    x2, h3, r3 = _tc_mid(acc2, denom, r2, g2, bt2, Wl3, Wr3, b3)
    (acc3,) = _SC_SCATTER(h3, src2, dst2, zrow)
    return _tc_final(acc3, denom, r3, g3, bt3, x1, x2, batch2d)


# speedup vs baseline: 12.9560x; 1.0607x over previous
import functools

import jax
import jax.numpy as jnp
from jax import lax
from jax.experimental import pallas as pl
from jax.experimental.pallas import tpu as pltpu
from jax.experimental.pallas import tpu_sc as plsc

N = 10000
E = 320000
D = 128
G = 16
EPS = 1e-5

NC = 2
NS = 16
K = 80
NP = 10240
ROWS_PER_TILE = NP // NS
CHUNKS = E // (NC * NS * K)
IB = 25
STAGES = CHUNKS // IB
F32 = jnp.float32


def _make_sc_scatter(with_count: bool):
    NB = 3 if with_count else 4
    A = NB - 2
    out_type = [jax.ShapeDtypeStruct((NC, NP, D), F32)]
    scratch = [
        pltpu.VMEM((IB, K), jnp.int32),
        pltpu.VMEM((IB, K), jnp.int32),
        pltpu.VMEM((NB, K, D), F32),
        pltpu.VMEM_SHARED((NP, D), F32),
        pltpu.SemaphoreType.DMA((NB,)),
        pltpu.SemaphoreType.DMA((NB,)),
    ]
    if with_count:
        out_type.append(jax.ShapeDtypeStruct((NC * NS, NP // 128, 128), F32))
        scratch.append(pltpu.VMEM((NP // 128, 128), F32))

    mesh = plsc.VectorSubcoreMesh(core_axis_name="c", subcore_axis_name="s")

    NZ = ROWS_PER_TILE // K

    def body(h, src_i, dst_i, zrow, *rest):
        if with_count:
            (out, cnt_out, src_v, dst_v, rows_v, acc_sp, sem, ssem,
             cnt_v) = rest
        else:
            (out, src_v, dst_v, rows_v, acc_sp, sem, ssem) = rest
        c = lax.axis_index("c")
        s = lax.axis_index("s")
        r0 = s * ROWS_PER_TILE
        pltpu.sync_copy(zrow, rows_v.at[0])
        if with_count:
            pltpu.sync_copy(zrow, cnt_v)

        def zfire(i, carry):
            pltpu.async_copy(rows_v.at[0], acc_sp.at[pl.ds(r0 + i * K, K)],
                             ssem.at[0])
            return carry

        lax.fori_loop(0, NZ, zfire, 0)

        def zdrain(i, carry):
            pltpu.make_async_copy(rows_v.at[0], acc_sp.at[pl.ds(r0, K)],
                                  ssem.at[0]).wait()
            return carry

        lax.fori_loop(0, NZ, zdrain, 0)
        w = c * NS + s
        plsc.subcore_barrier()
        ones16 = jnp.full((16,), 1.0, F32)

        def stage(t, carry):
            pltpu.sync_copy(src_i.at[w * STAGES + t], src_v)
            pltpu.sync_copy(dst_i.at[w * STAGES + t], dst_v)
            for p in range(A):
                pltpu.async_copy(h.at[src_v.at[p]], rows_v.at[p], sem.at[p])

            def step(j, c2):
                slot = lax.rem(j, NB)
                tgt = lax.rem(j + A, NB)

                @pl.when(j >= 2)
                def _drain():
                    pltpu.make_async_copy(zrow, rows_v.at[tgt],
                                          ssem.at[tgt]).wait()

                @pl.when(j + A < IB)
                def _fire():
                    pltpu.async_copy(h.at[src_v.at[j + A]],
                                     rows_v.at[tgt], sem.at[tgt])

                pltpu.make_async_copy(h.at[src_v.at[j]], rows_v.at[slot],
                                      sem.at[slot]).wait()
                pltpu.async_copy(rows_v.at[slot], acc_sp.at[dst_v.at[j]],
                                 ssem.at[slot], add=True)
                if with_count:
                    for g in range(K // 16):
                        dv = dst_v[j, pl.ds(g * 16, 16)]
                        plsc.addupdate_scatter(
                            cnt_v, [dv >> 7, dv & 127], ones16)
                return c2

            carry = lax.fori_loop(0, IB, step, carry)
            pltpu.make_async_copy(
                zrow, rows_v.at[(IB - 2) % NB],
                ssem.at[(IB - 2) % NB]).wait()
            pltpu.make_async_copy(
                zrow, rows_v.at[(IB - 1) % NB],
                ssem.at[(IB - 1) % NB]).wait()
            return carry

        lax.fori_loop(0, STAGES, stage, 0)
        if with_count:
            pltpu.sync_copy(cnt_v, cnt_out.at[w])
        plsc.subcore_barrier()

        def wstep(i, carry):
            slot = lax.rem(i, 2)

            @pl.when(i >= 2)
            def _drain():
                pltpu.make_async_copy(zrow, rows_v.at[slot],
                                      sem.at[slot]).wait()

            pltpu.sync_copy(acc_sp.at[pl.ds(r0 + i * K, K)], rows_v.at[slot])
            pltpu.async_copy(rows_v.at[slot], out.at[c, pl.ds(r0 + i * K, K)],
                             sem.at[slot])
            return carry

        lax.fori_loop(0, NZ, wstep, 0)
        pltpu.make_async_copy(zrow, rows_v.at[0], sem.at[0]).wait()
        pltpu.make_async_copy(zrow, rows_v.at[1], sem.at[1]).wait()

    return pl.kernel(
        body, out_type=out_type, mesh=mesh, scratch_types=scratch,
        compiler_params=pltpu.CompilerParams(needs_layout_passes=False))


_SC_SCATTER_CNT = _make_sc_scatter(True)
_SC_SCATTER = _make_sc_scatter(False)


def _tc_pre(x, wl, wr, b):
    def body(x_ref, wl_ref, wr_ref, b_ref, h_ref, r_ref):
        xv = x_ref[...]
        h_ref[...] = jnp.dot(xv, wl_ref[...], preferred_element_type=F32)
        r_ref[...] = (jnp.dot(xv, wr_ref[...], preferred_element_type=F32)
                      + b_ref[...])
    return pl.pallas_call(
        body,
        out_shape=[jax.ShapeDtypeStruct((N, D), F32),
                   jax.ShapeDtypeStruct((N, D), F32)],
    )(x, wl, wr, b.reshape(1, D))


def _bn_relu(y, gamma, beta):
    mu = jnp.mean(y, axis=0, keepdims=True)
    var = jnp.mean(y * y, axis=0, keepdims=True) - mu * mu
    xn = (y - mu) * lax.rsqrt(var + EPS)
    return jnp.maximum(gamma * xn + beta, 0.0)


def _tc_cnt(cnt_parts):
    def body(c_ref, o_ref):
        o_ref[...] = jnp.maximum(jnp.sum(c_ref[...], axis=0), 1.0)
    return pl.pallas_call(
        body,
        out_shape=jax.ShapeDtypeStruct((NP // 128, 128), F32),
    )(cnt_parts)


def _tc_mid(acc, denom, r, gamma, beta, wl, wr, b):
    def body(acc_ref, c_ref, r_ref, g_ref, bt_ref, wl_ref, wr_ref, b_ref,
             x_ref, h_ref, rn_ref):
        a = acc_ref[...]
        y = (a[0, :N] + a[1, :N]) / c_ref[...] + r_ref[...]
        xo = _bn_relu(y, g_ref[...], bt_ref[...])
        x_ref[...] = xo
        h_ref[...] = jnp.dot(xo, wl_ref[...], preferred_element_type=F32)
        rn_ref[...] = (jnp.dot(xo, wr_ref[...], preferred_element_type=F32)
                       + b_ref[...])
    return pl.pallas_call(
        body,
        out_shape=[jax.ShapeDtypeStruct((N, D), F32),
                   jax.ShapeDtypeStruct((N, D), F32),
                   jax.ShapeDtypeStruct((N, D), F32)],
    )(acc, denom, r, gamma.reshape(1, D), beta.reshape(1, D), wl, wr,
      b.reshape(1, D))


def _tc_final(acc, denom, r, gamma, beta, x1, x2, batch2d):
    def body(acc_ref, c_ref, r_ref, g_ref, bt_ref, x1_ref, x2_ref, b_ref,
             o_ref):
        a = acc_ref[...]
        y = (a[0, :N] + a[1, :N]) / c_ref[...] + r_ref[...]
        x3 = _bn_relu(y, g_ref[...], bt_ref[...])
        iota = lax.broadcasted_iota(jnp.int32, (G, N), 0)
        oh = (iota == b_ref[...]).astype(F32)
        p1 = jnp.dot(oh, x1_ref[...], preferred_element_type=F32)
        p2 = jnp.dot(oh, x2_ref[...], preferred_element_type=F32)
        p3 = jnp.dot(oh, x3, preferred_element_type=F32)
        counts = jnp.sum(oh, axis=1, keepdims=True)
        o_ref[...] = (jnp.concatenate([p1, p2, p3], axis=1)
                      / jnp.maximum(counts, 1.0))
    return pl.pallas_call(
        body,
        out_shape=jax.ShapeDtypeStruct((G, 3 * D), F32),
    )(acc, denom, r, gamma.reshape(1, D), beta.reshape(1, D), x1, x2,
      batch2d)


def kernel(x, edge_index, batch, Wl1, Wr1, b1, g1, bt1, Wl2, Wr2, b2, g2,
           bt2, Wl3, Wr3, b3, g3, bt3):
    src2 = edge_index[0].reshape(NC * NS * STAGES, IB, K)
    dst2 = edge_index[1].reshape(NC * NS * STAGES, IB, K)
    zrow = jnp.zeros((K, D), F32)
    batch2d = batch.reshape(1, N)

    h1, r1 = _tc_pre(x, Wl1, Wr1, b1)
    acc1, cnt_parts = _SC_SCATTER_CNT(h1, src2, dst2, zrow)
    denom = _tc_cnt(cnt_parts).reshape(NP, 1)[:N]
    x1, h2, r2 = _tc_mid(acc1, denom, r1, g1, bt1, Wl2, Wr2, b2)
    (acc2,) = _SC_SCATTER(h2, src2, dst2, zrow)
    x2, h3, r3 = _tc_mid(acc2, denom, r2, g2, bt2, Wl3, Wr3, b3)
    (acc3,) = _SC_SCATTER(h3, src2, dst2, zrow)
    return _tc_final(acc3, denom, r3, g3, bt3, x1, x2, batch2d)


# speedup vs baseline: 14.0087x; 1.0813x over previous
import functools

import jax
import jax.numpy as jnp
from jax import lax
from jax.experimental import pallas as pl
from jax.experimental.pallas import tpu as pltpu
from jax.experimental.pallas import tpu_sc as plsc

N = 10000
E = 320000
D = 128
G = 16
EPS = 1e-5

NC = 2
NS = 16
K = 80
NP = 10240
ROWS_PER_TILE = NP // NS
CHUNKS = E // (NC * NS * K)
IB = 25
STAGES = CHUNKS // IB
F32 = jnp.float32


def _make_sc_scatter(with_count: bool):
    NB = 3 if with_count else 4
    A = NB - 2
    IB = 5
    STAGES = CHUNKS // IB
    out_type = [jax.ShapeDtypeStruct((NC, NP, D), F32)]
    scratch = [
        pltpu.VMEM((2 * IB, K), jnp.int32),
        pltpu.VMEM((2 * IB, K), jnp.int32),
        pltpu.VMEM((NB, K, D), F32),
        pltpu.VMEM_SHARED((NP, D), F32),
        pltpu.SemaphoreType.DMA((NB,)),
        pltpu.SemaphoreType.DMA((NB,)),
        pltpu.SemaphoreType.DMA((2,)),
    ]
    if with_count:
        out_type.append(jax.ShapeDtypeStruct((NC * NS, NP // 128, 128), F32))
        scratch.append(pltpu.VMEM((NP // 128, 128), F32))

    mesh = plsc.VectorSubcoreMesh(core_axis_name="c", subcore_axis_name="s")

    NZ = ROWS_PER_TILE // K

    CH = CHUNKS

    def body(h, src_i, dst_i, zrow, *rest):
        if with_count:
            (out, cnt_out, src_v, dst_v, rows_v, acc_sp, sem, ssem, isem,
             cnt_v) = rest
        else:
            (out, src_v, dst_v, rows_v, acc_sp, sem, ssem, isem) = rest
        c = lax.axis_index("c")
        s = lax.axis_index("s")
        r0 = s * ROWS_PER_TILE
        pltpu.sync_copy(zrow, rows_v.at[0])
        if with_count:
            pltpu.sync_copy(zrow, cnt_v)

        def zfire(i, carry):
            pltpu.async_copy(rows_v.at[0], acc_sp.at[pl.ds(r0 + i * K, K)],
                             ssem.at[0])
            return carry

        lax.fori_loop(0, NZ, zfire, 0)

        def zdrain(i, carry):
            pltpu.make_async_copy(rows_v.at[0], acc_sp.at[pl.ds(r0, K)],
                                  ssem.at[0]).wait()
            return carry

        lax.fori_loop(0, NZ, zdrain, 0)
        w = c * NS + s
        plsc.subcore_barrier()
        ones16 = jnp.full((16,), 1.0, F32)

        def _slab_fire(t, slot):
            pltpu.async_copy(src_i.at[w * STAGES + t],
                             src_v.at[pl.ds(slot * IB, IB)], isem.at[slot])
            pltpu.async_copy(dst_i.at[w * STAGES + t],
                             dst_v.at[pl.ds(slot * IB, IB)], isem.at[slot])

        def _slab_wait(slot):
            pltpu.make_async_copy(src_i.at[w * STAGES],
                                  src_v.at[pl.ds(slot * IB, IB)],
                                  isem.at[slot]).wait()
            pltpu.make_async_copy(dst_i.at[w * STAGES],
                                  dst_v.at[pl.ds(slot * IB, IB)],
                                  isem.at[slot]).wait()

        _slab_fire(0, 0)
        _slab_wait(0)
        _slab_fire(1, 1)
        for p in range(A):
            pltpu.async_copy(h.at[src_v.at[p]], rows_v.at[p], sem.at[p])

        def step(j, c2):
            slot = lax.rem(j, NB)
            tgt = lax.rem(j + A, NB)
            jn = j + A

            @pl.when(j >= 2)
            def _drain():
                pltpu.make_async_copy(zrow, rows_v.at[tgt],
                                      ssem.at[tgt]).wait()

            @pl.when((lax.rem(jn, IB) == 0) & (jn < CH))
            def _slab_step():
                sl = jn // IB
                _slab_wait(lax.rem(sl, 2))

                @pl.when(sl + 1 < STAGES)
                def _pref():
                    _slab_fire(sl + 1, lax.rem(sl + 1, 2))

            @pl.when(jn < CH)
            def _fire():
                row = lax.rem(jn // IB, 2) * IB + lax.rem(jn, IB)
                pltpu.async_copy(h.at[src_v.at[row]], rows_v.at[tgt],
                                 sem.at[tgt])

            pltpu.make_async_copy(h.at[src_v.at[0]], rows_v.at[slot],
                                  sem.at[slot]).wait()
            drow = lax.rem(j // IB, 2) * IB + lax.rem(j, IB)
            pltpu.async_copy(rows_v.at[slot], acc_sp.at[dst_v.at[drow]],
                             ssem.at[slot], add=True)
            if with_count:
                for g in range(K // 16):
                    dv = dst_v[drow, pl.ds(g * 16, 16)]
                    plsc.addupdate_scatter(
                        cnt_v, [dv >> 7, dv & 127], ones16)
            return c2

        lax.fori_loop(0, CH, step, 0)
        pltpu.make_async_copy(
            zrow, rows_v.at[(CH - 2) % NB], ssem.at[(CH - 2) % NB]).wait()
        pltpu.make_async_copy(
            zrow, rows_v.at[(CH - 1) % NB], ssem.at[(CH - 1) % NB]).wait()

        if with_count:
            pltpu.sync_copy(cnt_v, cnt_out.at[w])
        plsc.subcore_barrier()

        def wstep(i, carry):
            slot = lax.rem(i, 2)

            @pl.when(i >= 2)
            def _drain():
                pltpu.make_async_copy(zrow, rows_v.at[slot],
                                      sem.at[slot]).wait()

            pltpu.sync_copy(acc_sp.at[pl.ds(r0 + i * K, K)], rows_v.at[slot])
            pltpu.async_copy(rows_v.at[slot], out.at[c, pl.ds(r0 + i * K, K)],
                             sem.at[slot])
            return carry

        lax.fori_loop(0, NZ, wstep, 0)
        pltpu.make_async_copy(zrow, rows_v.at[0], sem.at[0]).wait()
        pltpu.make_async_copy(zrow, rows_v.at[1], sem.at[1]).wait()

    return pl.kernel(
        body, out_type=out_type, mesh=mesh, scratch_types=scratch,
        compiler_params=pltpu.CompilerParams(needs_layout_passes=False))


_SC_SCATTER_CNT = _make_sc_scatter(True)
_SC_SCATTER = _make_sc_scatter(False)


def _tc_pre(x, wl, wr, b):
    def body(x_ref, wl_ref, wr_ref, b_ref, h_ref, r_ref):
        xv = x_ref[...]
        h_ref[...] = jnp.dot(xv, wl_ref[...], preferred_element_type=F32)
        r_ref[...] = (jnp.dot(xv, wr_ref[...], preferred_element_type=F32)
                      + b_ref[...])
    return pl.pallas_call(
        body,
        out_shape=[jax.ShapeDtypeStruct((N, D), F32),
                   jax.ShapeDtypeStruct((N, D), F32)],
    )(x, wl, wr, b.reshape(1, D))


def _bn_relu(y, gamma, beta):
    mu = jnp.mean(y, axis=0, keepdims=True)
    var = jnp.mean(y * y, axis=0, keepdims=True) - mu * mu
    xn = (y - mu) * lax.rsqrt(var + EPS)
    return jnp.maximum(gamma * xn + beta, 0.0)


def _tc_cnt(cnt_parts):
    def body(c_ref, o_ref):
        o_ref[...] = 1.0 / jnp.maximum(jnp.sum(c_ref[...], axis=0), 1.0)
    return pl.pallas_call(
        body,
        out_shape=jax.ShapeDtypeStruct((NP // 128, 128), F32),
    )(cnt_parts)


def _tc_mid(acc, denom, r, gamma, beta, wl, wr, b):
    def body(acc_ref, c_ref, r_ref, g_ref, bt_ref, wl_ref, wr_ref, b_ref,
             x_ref, h_ref, rn_ref):
        a = acc_ref[...]
        y = (a[0, :N] + a[1, :N]) * c_ref[...] + r_ref[...]
        xo = _bn_relu(y, g_ref[...], bt_ref[...])
        x_ref[...] = xo
        h_ref[...] = jnp.dot(xo, wl_ref[...], preferred_element_type=F32)
        rn_ref[...] = (jnp.dot(xo, wr_ref[...], preferred_element_type=F32)
                       + b_ref[...])
    return pl.pallas_call(
        body,
        out_shape=[jax.ShapeDtypeStruct((N, D), F32),
                   jax.ShapeDtypeStruct((N, D), F32),
                   jax.ShapeDtypeStruct((N, D), F32)],
    )(acc, denom, r, gamma.reshape(1, D), beta.reshape(1, D), wl, wr,
      b.reshape(1, D))


def _tc_final(acc, denom, r, gamma, beta, x1, x2, batch2d):
    def body(acc_ref, c_ref, r_ref, g_ref, bt_ref, x1_ref, x2_ref, b_ref,
             o_ref):
        a = acc_ref[...]
        y = (a[0, :N] + a[1, :N]) * c_ref[...] + r_ref[...]
        x3 = _bn_relu(y, g_ref[...], bt_ref[...])
        iota = lax.broadcasted_iota(jnp.int32, (G, N), 0)
        oh = (iota == b_ref[...]).astype(F32)
        p1 = jnp.dot(oh, x1_ref[...], preferred_element_type=F32)
        p2 = jnp.dot(oh, x2_ref[...], preferred_element_type=F32)
        p3 = jnp.dot(oh, x3, preferred_element_type=F32)
        counts = jnp.sum(oh, axis=1, keepdims=True)
        o_ref[...] = (jnp.concatenate([p1, p2, p3], axis=1)
                      / jnp.maximum(counts, 1.0))
    return pl.pallas_call(
        body,
        out_shape=jax.ShapeDtypeStruct((G, 3 * D), F32),
    )(acc, denom, r, gamma.reshape(1, D), beta.reshape(1, D), x1, x2,
      batch2d)


def kernel(x, edge_index, batch, Wl1, Wr1, b1, g1, bt1, Wl2, Wr2, b2, g2,
           bt2, Wl3, Wr3, b3, g3, bt3):
    src2 = edge_index[0].reshape(NC * NS * CHUNKS // 5, 5, K)
    dst2 = edge_index[1].reshape(NC * NS * CHUNKS // 5, 5, K)
    zrow = jnp.zeros((K, D), F32)
    batch2d = batch.reshape(1, N)

    h1, r1 = _tc_pre(x, Wl1, Wr1, b1)
    acc1, cnt_parts = _SC_SCATTER_CNT(h1, src2, dst2, zrow)
    denom = _tc_cnt(cnt_parts).reshape(NP, 1)[:N]
    x1, h2, r2 = _tc_mid(acc1, denom, r1, g1, bt1, Wl2, Wr2, b2)
    (acc2,) = _SC_SCATTER(h2, src2, dst2, zrow)
    x2, h3, r3 = _tc_mid(acc2, denom, r2, g2, bt2, Wl3, Wr3, b3)
    (acc3,) = _SC_SCATTER(h3, src2, dst2, zrow)
    return _tc_final(acc3, denom, r3, g3, bt3, x1, x2, batch2d)
